# initial kernel scaffold (unmeasured)
import functools

import jax
import jax.numpy as jnp
from jax import lax
from jax.experimental import pallas as pl
from jax.experimental.pallas import tpu as pltpu

E_GLOBAL = 8
E_LOCAL = 4
T_SHARD = 4096
D_MODEL = 2048
D_FF = 4096
CAP = 1280

T_ALL = 2 * T_SHARD


def _exchange(arrays, collective_id):
    n = len(arrays)

    def body(*refs):
        in_refs = refs[:n]
        out_refs = refs[n : 2 * n]
        send_sems, recv_sems = refs[2 * n], refs[2 * n + 1]
        my_x = lax.axis_index("x")
        my_y = lax.axis_index("y")
        peer = (my_x, 1 - my_y)

        barrier_sem = pltpu.get_barrier_semaphore()
        pl.semaphore_signal(
            barrier_sem, inc=1, device_id=peer,
            device_id_type=pl.DeviceIdType.MESH,
        )
        pl.semaphore_wait(barrier_sem, 1)

        rdmas = []
        for i in range(n):
            rdma = pltpu.make_async_remote_copy(
                src_ref=in_refs[i],
                dst_ref=out_refs[i],
                send_sem=send_sems.at[i],
                recv_sem=recv_sems.at[i],
                device_id=peer,
                device_id_type=pl.DeviceIdType.MESH,
            )
            rdma.start()
            rdmas.append(rdma)
        for rdma in rdmas:
            rdma.wait()

    out = pl.pallas_call(
        body,
        out_shape=[jax.ShapeDtypeStruct(a.shape, a.dtype) for a in arrays],
        in_specs=[pl.BlockSpec(memory_space=pltpu.MemorySpace.ANY)] * n,
        out_specs=[pl.BlockSpec(memory_space=pltpu.MemorySpace.ANY)] * n,
        scratch_shapes=[
            pltpu.SemaphoreType.DMA((n,)),
            pltpu.SemaphoreType.DMA((n,)),
        ],
        compiler_params=pltpu.CompilerParams(collective_id=collective_id),
    )(*arrays)
    return out


T_TILE = 256
F_TILE = 512


def _moe_body(x_ref, w1_ref, w2_ref, out_ref):
    f = pl.program_id(2)
    h = jnp.maximum(
        jnp.dot(x_ref[0], w1_ref[0], preferred_element_type=jnp.float32), 0.0
    )
    contrib = jnp.dot(h, w2_ref[0], preferred_element_type=jnp.float32)

    @pl.when(f == 0)
    def _():
        out_ref[0] = contrib

    @pl.when(f > 0)
    def _():
        out_ref[0] = out_ref[0] + contrib


def _moe(xg, W1, W2):
    grid = (E_LOCAL, CAP // T_TILE, D_FF // F_TILE)
    return pl.pallas_call(
        _moe_body,
        grid=grid,
        out_shape=jax.ShapeDtypeStruct((E_LOCAL, CAP, D_MODEL), jnp.float32),
        in_specs=[
            pl.BlockSpec((1, T_TILE, D_MODEL), lambda e, t, f: (e, t, 0)),
            pl.BlockSpec((1, D_MODEL, F_TILE), lambda e, t, f: (e, 0, f)),
            pl.BlockSpec((1, F_TILE, D_MODEL), lambda e, t, f: (e, f, 0)),
        ],
        out_specs=pl.BlockSpec((1, T_TILE, D_MODEL), lambda e, t, f: (e, t, 0)),
        compiler_params=pltpu.CompilerParams(
            dimension_semantics=("parallel", "parallel", "arbitrary"),
        ),
    )(xg, W1, W2)


def kernel(x, assign, W1, W2):
    my_y = lax.axis_index("y")
    e0 = my_y * E_LOCAL

    assign2d = assign.reshape(32, 128)
    x_other, assign_other2d = _exchange([x, assign2d], collective_id=0)
    assign_other = assign_other2d.reshape(T_SHARD)

    x_all = jnp.concatenate([x, x_other], axis=0)
    assign_all = jnp.concatenate([assign, assign_other])

    local_e = assign_all - e0
    onehot = local_e[:, None] == jnp.arange(E_LOCAL, dtype=jnp.int32)[None, :]
    pos_in_e = jnp.cumsum(onehot.astype(jnp.int32), axis=0) - 1
    counts = jnp.sum(onehot, axis=0)
    my_pos = jnp.sum(pos_in_e * onehot, axis=1)
    is_local = (local_e >= 0) & (local_e < E_LOCAL)
    valid = is_local & (my_pos < CAP)
    e_clamped = jnp.clip(local_e, 0, E_LOCAL - 1)
    dest_slot = jnp.where(valid, e_clamped * CAP + my_pos, E_LOCAL * CAP)
    token_ids = jnp.arange(T_ALL, dtype=jnp.int32)
    slot_to_token = (
        jnp.zeros(E_LOCAL * CAP + 1, jnp.int32).at[dest_slot].set(token_ids)
    )[: E_LOCAL * CAP]

    xg = x_all[slot_to_token].reshape(E_LOCAL, CAP, D_MODEL)

    yg = _moe(xg, W1, W2)

    slot_valid = (
        jnp.arange(CAP, dtype=jnp.int32)[None, :] < counts[:, None]
    ).reshape(-1)
    dest_token = jnp.where(slot_valid, slot_to_token, T_ALL)
    out_all = (
        jnp.zeros((T_ALL + 1, D_MODEL), jnp.float32)
        .at[dest_token]
        .set(yg.reshape(-1, D_MODEL))
    )
    out_mine = out_all[:T_SHARD]
    out_theirs = out_all[T_SHARD:T_ALL]

    (peer_part,) = _exchange([out_theirs], collective_id=1)
    return out_mine + peer_part


# baseline (device time: 2157165 ns/iter reference)
import functools

import jax
import jax.numpy as jnp
from jax import lax
from jax.experimental import pallas as pl
from jax.experimental.pallas import tpu as pltpu

E_GLOBAL = 8
E_LOCAL = 4
T_SHARD = 4096
D_MODEL = 2048
D_FF = 4096
CAP = 1280

T_ALL = 2 * T_SHARD


def _exchange(arrays, collective_id):
    n = len(arrays)

    def body(*refs):
        in_refs = refs[:n]
        out_refs = refs[n : 2 * n]
        send_sems, recv_sems = refs[2 * n], refs[2 * n + 1]
        my_x = lax.axis_index("x")
        my_y = lax.axis_index("y")
        peer = (my_x, 1 - my_y)

        barrier_sem = pltpu.get_barrier_semaphore()
        pl.semaphore_signal(
            barrier_sem, inc=1, device_id=peer,
            device_id_type=pl.DeviceIdType.MESH,
        )
        pl.semaphore_wait(barrier_sem, 1)

        rdmas = []
        for i in range(n):
            rdma = pltpu.make_async_remote_copy(
                src_ref=in_refs[i],
                dst_ref=out_refs[i],
                send_sem=send_sems.at[i],
                recv_sem=recv_sems.at[i],
                device_id=peer,
                device_id_type=pl.DeviceIdType.MESH,
            )
            rdma.start()
            rdmas.append(rdma)
        for rdma in rdmas:
            rdma.wait()

    out = pl.pallas_call(
        body,
        out_shape=[jax.ShapeDtypeStruct(a.shape, a.dtype) for a in arrays],
        in_specs=[pl.BlockSpec(memory_space=pl.ANY)] * n,
        out_specs=[pl.BlockSpec(memory_space=pl.ANY)] * n,
        scratch_shapes=[
            pltpu.SemaphoreType.DMA((n,)),
            pltpu.SemaphoreType.DMA((n,)),
        ],
        compiler_params=pltpu.CompilerParams(collective_id=collective_id),
    )(*arrays)
    return out


T_TILE = 256
F_TILE = 512


def _moe_body(x_ref, w1_ref, w2_ref, out_ref):
    f = pl.program_id(2)
    h = jnp.maximum(
        jnp.dot(x_ref[0], w1_ref[0], preferred_element_type=jnp.float32), 0.0
    )
    contrib = jnp.dot(h, w2_ref[0], preferred_element_type=jnp.float32)

    @pl.when(f == 0)
    def _():
        out_ref[0] = contrib

    @pl.when(f > 0)
    def _():
        out_ref[0] = out_ref[0] + contrib


def _moe(xg, W1, W2):
    grid = (E_LOCAL, CAP // T_TILE, D_FF // F_TILE)
    return pl.pallas_call(
        _moe_body,
        grid=grid,
        out_shape=jax.ShapeDtypeStruct((E_LOCAL, CAP, D_MODEL), jnp.float32),
        in_specs=[
            pl.BlockSpec((1, T_TILE, D_MODEL), lambda e, t, f: (e, t, 0)),
            pl.BlockSpec((1, D_MODEL, F_TILE), lambda e, t, f: (e, 0, f)),
            pl.BlockSpec((1, F_TILE, D_MODEL), lambda e, t, f: (e, f, 0)),
        ],
        out_specs=pl.BlockSpec((1, T_TILE, D_MODEL), lambda e, t, f: (e, t, 0)),
        compiler_params=pltpu.CompilerParams(
            dimension_semantics=("parallel", "parallel", "arbitrary"),
        ),
    )(xg, W1, W2)


def kernel(x, assign, W1, W2):
    my_y = lax.axis_index("y")
    e0 = my_y * E_LOCAL

    assign2d = assign.reshape(32, 128)
    x_other, assign_other2d = _exchange([x, assign2d], collective_id=0)
    assign_other = assign_other2d.reshape(T_SHARD)

    x_all = jnp.concatenate([x, x_other], axis=0)
    assign_all = jnp.concatenate([assign, assign_other])

    local_e = assign_all - e0
    onehot = local_e[:, None] == jnp.arange(E_LOCAL, dtype=jnp.int32)[None, :]
    pos_in_e = jnp.cumsum(onehot.astype(jnp.int32), axis=0) - 1
    counts = jnp.sum(onehot, axis=0)
    my_pos = jnp.sum(pos_in_e * onehot, axis=1)
    is_local = (local_e >= 0) & (local_e < E_LOCAL)
    valid = is_local & (my_pos < CAP)
    e_clamped = jnp.clip(local_e, 0, E_LOCAL - 1)
    dest_slot = jnp.where(valid, e_clamped * CAP + my_pos, E_LOCAL * CAP)
    token_ids = jnp.arange(T_ALL, dtype=jnp.int32)
    slot_to_token = (
        jnp.zeros(E_LOCAL * CAP + 1, jnp.int32).at[dest_slot].set(token_ids)
    )[: E_LOCAL * CAP]

    xg = x_all[slot_to_token].reshape(E_LOCAL, CAP, D_MODEL)

    yg = _moe(xg, W1, W2)

    slot_valid = (
        jnp.arange(CAP, dtype=jnp.int32)[None, :] < counts[:, None]
    ).reshape(-1)
    dest_token = jnp.where(slot_valid, slot_to_token, T_ALL)
    out_all = (
        jnp.zeros((T_ALL + 1, D_MODEL), jnp.float32)
        .at[dest_token]
        .set(yg.reshape(-1, D_MODEL))
    )
    out_mine = out_all[:T_SHARD]
    out_theirs = out_all[T_SHARD:T_ALL]

    (peer_part,) = _exchange([out_theirs], collective_id=1)
    return out_mine + peer_part


# device time: 1147433 ns/iter; 1.8800x vs baseline; 1.8800x over previous
import jax
import jax.numpy as jnp
from jax import lax
from jax.experimental import pallas as pl
from jax.experimental.pallas import tpu as pltpu

E_LOCAL = 4
T_SHARD = 4096
D_MODEL = 2048
D_FF = 4096
CAP = 1280
RCAP = 2304

T_ALL = 2 * T_SHARD
N_SLOT = E_LOCAL * CAP
A_ROWS = 32
N_CHUNK = 4
CHUNK = RCAP // N_CHUNK


def _peer():
    return (lax.axis_index("x"), 1 - lax.axis_index("y"))


def _barrier(peer):
    barrier_sem = pltpu.get_barrier_semaphore()
    pl.semaphore_signal(
        barrier_sem, inc=1, device_id=peer,
        device_id_type=pl.DeviceIdType.MESH,
    )
    pl.semaphore_wait(barrier_sem, 1)


def _dispatch(x, assign2d, send_idx):

    def body(x_ref, a_ref, sidx_ref, xrecv_ref, aall_ref,
             comp_ref, lsem, gsem, send_sems, recv_sems):
        peer = _peer()
        _barrier(peer)

        cp_a = pltpu.make_async_copy(
            a_ref, aall_ref.at[pl.ds(0, A_ROWS)], lsem
        )
        rd_a = pltpu.make_async_remote_copy(
            src_ref=a_ref,
            dst_ref=aall_ref.at[pl.ds(A_ROWS, A_ROWS)],
            send_sem=send_sems.at[0],
            recv_sem=recv_sems.at[0],
            device_id=peer,
            device_id_type=pl.DeviceIdType.MESH,
        )
        cp_a.start()
        rd_a.start()

        def issue(i, c):
            row = jnp.minimum(sidx_ref[i], T_SHARD - 1)
            pltpu.make_async_copy(
                x_ref.at[pl.ds(row, 1)], comp_ref.at[pl.ds(i, 1)], gsem
            ).start()
            return c

        def drain(i, c):
            pltpu.make_async_copy(
                x_ref.at[pl.ds(0, 1)], comp_ref.at[pl.ds(0, 1)], gsem
            ).wait()
            return c

        rds = []
        for k in range(N_CHUNK):
            lax.fori_loop(k * CHUNK, (k + 1) * CHUNK, issue, 0)
            lax.fori_loop(0, CHUNK, drain, 0)
            rd = pltpu.make_async_remote_copy(
                src_ref=comp_ref.at[pl.ds(k * CHUNK, CHUNK)],
                dst_ref=xrecv_ref.at[pl.ds(k * CHUNK, CHUNK)],
                send_sem=send_sems.at[1 + k],
                recv_sem=recv_sems.at[1 + k],
                device_id=peer,
                device_id_type=pl.DeviceIdType.MESH,
            )
            rd.start()
            rds.append(rd)
        cp_a.wait()
        rd_a.wait()
        for rd in rds:
            rd.wait()

    return pl.pallas_call(
        body,
        out_shape=[
            jax.ShapeDtypeStruct((RCAP, x.shape[1]), x.dtype),
            jax.ShapeDtypeStruct((2 * A_ROWS, 128), assign2d.dtype),
        ],
        in_specs=[
            pl.BlockSpec(memory_space=pl.ANY),
            pl.BlockSpec(memory_space=pl.ANY),
            pl.BlockSpec(memory_space=pltpu.MemorySpace.SMEM),
        ],
        out_specs=[pl.BlockSpec(memory_space=pl.ANY)] * 2,
        scratch_shapes=[
            pltpu.VMEM((RCAP, x.shape[1]), x.dtype),
            pltpu.SemaphoreType.DMA,
            pltpu.SemaphoreType.DMA,
            pltpu.SemaphoreType.DMA((1 + N_CHUNK,)),
            pltpu.SemaphoreType.DMA((1 + N_CHUNK,)),
        ],
        compiler_params=pltpu.CompilerParams(collective_id=0),
    )(x, assign2d, send_idx)


def _combine(yg, comp_slot):

    def body(yg_ref, cslot_ref, recv_ref, comp_ref, gsem, ssem, rsem):
        peer = _peer()
        _barrier(peer)

        def issue(i, c):
            row = jnp.minimum(cslot_ref[i], N_SLOT - 1)
            pltpu.make_async_copy(
                yg_ref.at[pl.ds(row, 1)], comp_ref.at[pl.ds(i, 1)], gsem
            ).start()
            return c

        def drain(i, c):
            pltpu.make_async_copy(
                yg_ref.at[pl.ds(0, 1)], comp_ref.at[pl.ds(0, 1)], gsem
            ).wait()
            return c

        rds = []
        for k in range(N_CHUNK):
            lax.fori_loop(k * CHUNK, (k + 1) * CHUNK, issue, 0)
            lax.fori_loop(0, CHUNK, drain, 0)
            rd = pltpu.make_async_remote_copy(
                src_ref=comp_ref.at[pl.ds(k * CHUNK, CHUNK)],
                dst_ref=recv_ref.at[pl.ds(k * CHUNK, CHUNK)],
                send_sem=ssem.at[k],
                recv_sem=rsem.at[k],
                device_id=peer,
                device_id_type=pl.DeviceIdType.MESH,
            )
            rd.start()
            rds.append(rd)
        for rd in rds:
            rd.wait()

    return pl.pallas_call(
        body,
        out_shape=jax.ShapeDtypeStruct((RCAP, yg.shape[1]), yg.dtype),
        in_specs=[
            pl.BlockSpec(memory_space=pl.ANY),
            pl.BlockSpec(memory_space=pltpu.MemorySpace.SMEM),
        ],
        out_specs=pl.BlockSpec(memory_space=pl.ANY),
        scratch_shapes=[
            pltpu.VMEM((RCAP, yg.shape[1]), yg.dtype),
            pltpu.SemaphoreType.DMA,
            pltpu.SemaphoreType.DMA((N_CHUNK,)),
            pltpu.SemaphoreType.DMA((N_CHUNK,)),
        ],
        compiler_params=pltpu.CompilerParams(collective_id=1),
    )(yg, comp_slot)


R_TILE = 256


def _gather_rows(src, idx, n_out):
    n_src, d = src.shape

    def body(src_ref, idx_ref, out_ref, sem):
        t = pl.program_id(0)

        def issue(i, c):
            row = jnp.minimum(idx_ref[t * R_TILE + i], n_src - 1)
            pltpu.make_async_copy(
                src_ref.at[pl.ds(row, 1)], out_ref.at[pl.ds(i, 1)], sem
            ).start()
            return c

        lax.fori_loop(0, R_TILE, issue, 0)

        def drain(i, c):
            pltpu.make_async_copy(
                src_ref.at[pl.ds(0, 1)], out_ref.at[pl.ds(0, 1)], sem
            ).wait()
            return c

        lax.fori_loop(0, R_TILE, drain, 0)

    return pl.pallas_call(
        body,
        grid=(n_out // R_TILE,),
        out_shape=jax.ShapeDtypeStruct((n_out, d), src.dtype),
        in_specs=[
            pl.BlockSpec(memory_space=pl.ANY),
            pl.BlockSpec(memory_space=pltpu.MemorySpace.SMEM),
        ],
        out_specs=pl.BlockSpec((R_TILE, d), lambda t: (t, 0)),
        scratch_shapes=[pltpu.SemaphoreType.DMA],
    )(src, idx)


def _gather_two(src1, src2, idx, n_out):
    n1, d = src1.shape
    n2 = src2.shape[0]

    def body(s1_ref, s2_ref, idx_ref, out_ref, sem):
        t = pl.program_id(0)

        def issue(i, c):
            v = idx_ref[t * R_TILE + i]

            @pl.when(v < n1)
            def _():
                pltpu.make_async_copy(
                    s1_ref.at[pl.ds(v, 1)], out_ref.at[pl.ds(i, 1)], sem
                ).start()

            @pl.when(v >= n1)
            def _():
                row = jnp.minimum(v - n1, n2 - 1)
                pltpu.make_async_copy(
                    s2_ref.at[pl.ds(row, 1)], out_ref.at[pl.ds(i, 1)], sem
                ).start()

            return c

        lax.fori_loop(0, R_TILE, issue, 0)

        def drain(i, c):
            pltpu.make_async_copy(
                s1_ref.at[pl.ds(0, 1)], out_ref.at[pl.ds(0, 1)], sem
            ).wait()
            return c

        lax.fori_loop(0, R_TILE, drain, 0)

    return pl.pallas_call(
        body,
        grid=(n_out // R_TILE,),
        out_shape=jax.ShapeDtypeStruct((n_out, d), src1.dtype),
        in_specs=[
            pl.BlockSpec(memory_space=pl.ANY),
            pl.BlockSpec(memory_space=pl.ANY),
            pl.BlockSpec(memory_space=pltpu.MemorySpace.SMEM),
        ],
        out_specs=pl.BlockSpec((R_TILE, d), lambda t: (t, 0)),
        scratch_shapes=[pltpu.SemaphoreType.DMA],
    )(src1, src2, idx)


T_TILE = 640
F_TILE = 512


N_F = D_FF // F_TILE


def _moe_body(x_ref, w1_ref, w2_ref, out_ref, acc_ref):
    f = pl.program_id(2)
    h = jnp.maximum(
        jnp.dot(
            x_ref[0].astype(jnp.float32),
            w1_ref[0],
            preferred_element_type=jnp.float32,
        ),
        0.0,
    )
    contrib = jnp.dot(h, w2_ref[0], preferred_element_type=jnp.float32)

    @pl.when(f == 0)
    def _():
        acc_ref[...] = contrib

    @pl.when(f > 0)
    def _():
        acc_ref[...] = acc_ref[...] + contrib

    @pl.when(f == N_F - 1)
    def _():
        out_ref[0] = acc_ref[...].astype(out_ref.dtype)


def _moe(xg, W1, W2):
    grid = (E_LOCAL, CAP // T_TILE, N_F)
    return pl.pallas_call(
        _moe_body,
        grid=grid,
        out_shape=jax.ShapeDtypeStruct((E_LOCAL, CAP, D_MODEL), jnp.float32),
        in_specs=[
            pl.BlockSpec((1, T_TILE, D_MODEL), lambda e, t, f: (e, t, 0)),
            pl.BlockSpec((1, D_MODEL, F_TILE), lambda e, t, f: (e, 0, f)),
            pl.BlockSpec((1, F_TILE, D_MODEL), lambda e, t, f: (e, f, 0)),
        ],
        out_specs=pl.BlockSpec((1, T_TILE, D_MODEL), lambda e, t, f: (e, t, 0)),
        scratch_shapes=[pltpu.VMEM((T_TILE, D_MODEL), jnp.float32)],
        compiler_params=pltpu.CompilerParams(
            dimension_semantics=("parallel", "parallel", "arbitrary"),
            vmem_limit_bytes=56 * 1024 * 1024,
        ),
    )(xg, W1, W2)


def _presync(a):

    def body(a_ref, o_ref, sem):
        _barrier(_peer())
        cp = pltpu.make_async_copy(a_ref, o_ref, sem)
        cp.start()
        cp.wait()

    return pl.pallas_call(
        body,
        out_shape=jax.ShapeDtypeStruct(a.shape, a.dtype),
        in_specs=[pl.BlockSpec(memory_space=pl.ANY)],
        out_specs=pl.BlockSpec(memory_space=pl.ANY),
        scratch_shapes=[pltpu.SemaphoreType.DMA],
        compiler_params=pltpu.CompilerParams(collective_id=2),
    )(a)


def kernel(x, assign, W1, W2):
    my_y = lax.axis_index("y")
    e0 = my_y * E_LOCAL
    i32 = jnp.int32

    is_mine_t = (assign >= e0) & (assign < e0 + E_LOCAL)
    to_peer = ~is_mine_t
    p = jnp.cumsum(to_peer.astype(i32)) - 1
    tok = jnp.arange(T_SHARD, dtype=i32)
    send_idx = (
        jnp.zeros(RCAP + 1, i32)
        .at[jnp.where(to_peer & (p < RCAP), p, RCAP)]
        .set(tok)
    )[:RCAP]

    assign2d = assign.reshape(A_ROWS, 128)
    xrecv, aall = _dispatch(x, assign2d, send_idx)
    assign_other = aall.reshape(T_ALL)[T_SHARD:]

    assign_all = jnp.concatenate([assign, assign_other])
    local_e = assign_all - e0
    onehot = local_e[:, None] == jnp.arange(E_LOCAL, dtype=i32)[None, :]
    pos_in_e = jnp.cumsum(onehot.astype(i32), axis=0) - 1
    my_pos = jnp.sum(pos_in_e * onehot, axis=1)
    is_local = (local_e >= 0) & (local_e < E_LOCAL)

    to_me_o = is_local[T_SHARD:]
    q = jnp.cumsum(to_me_o.astype(i32)) - 1
    row_of_token = jnp.concatenate(
        [tok, jnp.where(to_me_o, T_SHARD + jnp.minimum(q, RCAP - 1), 0)]
    )

    valid = is_local & (my_pos < CAP)
    valid = valid & jnp.concatenate(
        [jnp.ones(T_SHARD, bool), q < RCAP]
    )
    e_clamped = jnp.clip(local_e, 0, E_LOCAL - 1)
    dest_slot = jnp.where(valid, e_clamped * CAP + my_pos, N_SLOT)
    slot_to_row = (
        jnp.zeros(N_SLOT + 1, i32)
        .at[dest_slot]
        .set(row_of_token)
    )[:N_SLOT]

    xg = _gather_two(x, xrecv, slot_to_row, N_SLOT).reshape(
        E_LOCAL, CAP, D_MODEL
    )
    yg = _moe(xg, W1, W2).reshape(N_SLOT, D_MODEL)

    comp_slot = (
        jnp.zeros(RCAP + 1, i32)
        .at[jnp.where(to_me_o & (q < RCAP), q, RCAP)]
        .set(dest_slot[T_SHARD:])
    )[:RCAP]
    recv_comp = _combine(yg, comp_slot)

    merged = jnp.where(
        is_mine_t & (my_pos[:T_SHARD] < CAP),
        dest_slot[:T_SHARD],
        N_SLOT + jnp.minimum(p, RCAP - 1),
    )
    return _gather_two(yg, recv_comp, merged, T_SHARD)
